# baseline (speedup 1.0000x reference)
import functools

import jax
import jax.numpy as jnp
from jax import lax
from jax.experimental import pallas as pl
from jax.experimental.pallas import tpu as pltpu
from jax.experimental.pallas import tpu_sc as plsc

N = 10000
E = 320000
H = 128
HH = H // 2
NPAD = 10240
ROWS_PER_TILE = NPAD // 16
CH = 80
LANES = 16
EPT = E // 16
NCHUNKS = EPT // CH

_f32 = jnp.float32


def _fill_const(ref, nrows, ncols, val):
    v16 = jnp.full((LANES,), val, _f32)

    def body(r, carry):
        for cc in range(ncols // LANES):
            ref[r, pl.ds(cc * LANES, LANES)] = v16
        return carry

    lax.fori_loop(0, nrows, body, None)


def _zero_acc(acc, zbuf, tid, zrows):
    def body(j, carry):
        pltpu.sync_copy(zbuf,
                        acc.at[pl.ds(tid * ROWS_PER_TILE + j * zrows, zrows)])
        return carry

    lax.fori_loop(0, ROWS_PER_TILE // zrows, body, None)


NBUF = 5
NGRP = NCHUNKS // NBUF
CHROWS = E // CH
CHA = 40
CHAROWS = E // CHA
NCHUNKS_A = EPT // CHA
NGRP_A = NCHUNKS_A // NBUF
ZROWS = 64


def _sc_pass_a(ea_hbm, dsta_hbm, eaL_out, eaR_out, degc_out,
               idx_d, erows, ones_v, zbuf, zd, eacc, degacc,
               sem_i0, sem_i1, sems_r, sems_w, sem_dg0, sem_dg1):
    cid = lax.axis_index("c")
    tid = lax.axis_index("s")

    _fill_const(zbuf, ZROWS, HH, 0.0)
    _fill_const(zd, 128, LANES, 0.0)
    _fill_const(ones_v, CHA, LANES, 1.0)
    _zero_acc(eacc, zbuf, tid, ZROWS)

    @pl.when(cid == 0)
    def _():
        def zero_deg(j, carry):
            pltpu.sync_copy(zd, degacc.at[pl.ds(tid * ROWS_PER_TILE + j * 128, 128)])
            return carry

        lax.fori_loop(0, ROWS_PER_TILE // 128, zero_deg, None)

    plsc.subcore_barrier()

    sem_i = (sem_i0, sem_i1)
    sem_dg = (sem_dg0, sem_dg1)
    irow0 = tid * NCHUNKS_A

    def fire_idx(g, s):
        pltpu.async_copy(dsta_hbm.at[pl.ds(irow0 + g * NBUF, NBUF)],
                         idx_d.at[s], sem_i[s])

    def drain_scatters(s):
        for b in range(NBUF):
            pltpu.make_async_copy(erows.at[s, b], eacc.at[pl.ds(0, CHA)],
                                  sems_w[s][b]).wait()

        @pl.when(cid == 0)
        def _():
            for b in range(NBUF):
                pltpu.make_async_copy(ones_v, degacc.at[pl.ds(0, CHA)],
                                      sem_dg[s]).wait()

    def group(g, s):
        pltpu.make_async_copy(dsta_hbm.at[pl.ds(irow0, NBUF)],
                              idx_d.at[s], sem_i[s]).wait()
        for b in range(NBUF):
            gch = tid * EPT + (g * NBUF + b) * CHA

            @pl.when(cid == 0)
            def _(gch=gch, b=b):
                pltpu.async_copy(ea_hbm.at[pl.ds(gch, CHA), pl.ds(0, HH)],
                                 erows.at[s, b], sems_r[b])
                pltpu.async_copy(ones_v, degacc.at[idx_d.at[s, b]], sem_dg[s],
                                 add=True)

            @pl.when(cid == 1)
            def _(gch=gch, b=b):
                pltpu.async_copy(ea_hbm.at[pl.ds(gch, CHA), pl.ds(HH, HH)],
                                 erows.at[s, b], sems_r[b])

        for b in range(NBUF):
            pltpu.make_async_copy(ea_hbm.at[pl.ds(0, CHA), pl.ds(0, HH)],
                                  erows.at[s, b], sems_r[b]).wait()
            pltpu.async_copy(erows.at[s, b], eacc.at[idx_d.at[s, b]],
                             sems_w[s][b], add=True)

        @pl.when(g >= 1)
        def _():
            drain_scatters(1 - s)

        @pl.when(g + 1 < NGRP_A)
        def _():
            fire_idx(g + 1, 1 - s)

    fire_idx(0, 0)

    def outer(o, carry):
        group(2 * o, 0)
        group(2 * o + 1, 1)
        return carry

    lax.fori_loop(0, NGRP_A // 2, outer, None)
    drain_scatters((NGRP_A - 1) % 2)
    plsc.subcore_barrier()
    sl = pl.ds(tid * ROWS_PER_TILE, ROWS_PER_TILE)

    @pl.when(cid == 0)
    def _():
        pltpu.sync_copy(eacc.at[sl], eaL_out.at[sl])
        pltpu.sync_copy(degacc.at[sl], degc_out.at[sl])

    @pl.when(cid == 1)
    def _():
        pltpu.sync_copy(eacc.at[sl], eaR_out.at[sl])


def _sc_pass_b(hL_hbm, hR_hbm, src2_hbm, dst2_hbm, sL_out, sR_out,
               idx_s, idx_d, rows, zbuf, acc,
               sem_is0, sem_is1, sem_id0, sem_id1, sems_g, sems_w):
    cid = lax.axis_index("c")
    tid = lax.axis_index("s")

    _fill_const(zbuf, ZROWS, HH, 0.0)
    _zero_acc(acc, zbuf, tid, ZROWS)
    plsc.subcore_barrier()

    sem_is = (sem_is0, sem_is1)
    sem_id = (sem_id0, sem_id1)
    irow0 = tid * NCHUNKS

    def fire_idx(g, s):
        sl_i = pl.ds(irow0 + g * NBUF, NBUF)
        pltpu.async_copy(src2_hbm.at[sl_i], idx_s.at[s], sem_is[s])
        pltpu.async_copy(dst2_hbm.at[sl_i], idx_d.at[s], sem_id[s])

    def drain_scatters(s):
        for b in range(NBUF):
            pltpu.make_async_copy(rows.at[s, b], acc.at[pl.ds(0, CH)],
                                  sems_w[s][b]).wait()

    def group(g, s):
        sl_i = pl.ds(irow0, NBUF)
        pltpu.make_async_copy(src2_hbm.at[sl_i], idx_s.at[s], sem_is[s]).wait()
        pltpu.make_async_copy(dst2_hbm.at[sl_i], idx_d.at[s], sem_id[s]).wait()
        for b in range(NBUF):
            @pl.when(cid == 0)
            def _(b=b):
                pltpu.async_copy(hL_hbm.at[idx_s.at[s, b]], rows.at[s, b],
                                 sems_g[b])

            @pl.when(cid == 1)
            def _(b=b):
                pltpu.async_copy(hR_hbm.at[idx_s.at[s, b]], rows.at[s, b],
                                 sems_g[b])

        for b in range(NBUF):
            pltpu.make_async_copy(hL_hbm.at[idx_s.at[s, b]], rows.at[s, b],
                                  sems_g[b]).wait()
            pltpu.async_copy(rows.at[s, b], acc.at[idx_d.at[s, b]],
                             sems_w[s][b], add=True)

        @pl.when(g >= 1)
        def _():
            drain_scatters(1 - s)

        @pl.when(g + 1 < NGRP)
        def _():
            fire_idx(g + 1, 1 - s)

    fire_idx(0, 0)

    def outer(o, carry):
        group(2 * o, 0)
        group(2 * o + 1, 1)
        return carry

    lax.fori_loop(0, NGRP // 2, outer, None)
    drain_scatters((NGRP - 1) % 2)
    plsc.subcore_barrier()
    sl = pl.ds(tid * ROWS_PER_TILE, ROWS_PER_TILE)

    @pl.when(cid == 0)
    def _():
        pltpu.sync_copy(acc.at[sl], sL_out.at[sl])

    @pl.when(cid == 1)
    def _():
        pltpu.sync_copy(acc.at[sl], sR_out.at[sl])


@functools.lru_cache(maxsize=1)
def _sc_calls():
    mesh = plsc.VectorSubcoreMesh(core_axis_name="c", subcore_axis_name="s")
    params = pltpu.CompilerParams(use_tc_tiling_on_sc=False)
    half = jax.ShapeDtypeStruct((NPAD, HH), _f32)
    dma = pltpu.SemaphoreType.DMA
    sca = pl.kernel(
        _sc_pass_a,
        out_type=(half, half, jax.ShapeDtypeStruct((NPAD, LANES), _f32)),
        mesh=mesh,
        compiler_params=params,
        scratch_types=[
            pltpu.VMEM((2, NBUF, CHA), jnp.int32),
            pltpu.VMEM((2, NBUF, CHA, HH), _f32),
            pltpu.VMEM((CHA, LANES), _f32),
            pltpu.VMEM((ZROWS, HH), _f32),
            pltpu.VMEM((128, LANES), _f32),
            pltpu.VMEM_SHARED((NPAD, HH), _f32),
            pltpu.VMEM_SHARED((NPAD, LANES), _f32),
            dma, dma,
            [dma] * NBUF,
            [[dma] * NBUF, [dma] * NBUF],
            dma, dma,
        ],
    )
    scb = pl.kernel(
        _sc_pass_b,
        out_type=(half, half),
        mesh=mesh,
        compiler_params=params,
        scratch_types=[
            pltpu.VMEM((2, NBUF, CH), jnp.int32),
            pltpu.VMEM((2, NBUF, CH), jnp.int32),
            pltpu.VMEM((2, NBUF, CH, HH), _f32),
            pltpu.VMEM((ZROWS, HH), _f32),
            pltpu.VMEM_SHARED((NPAD, HH), _f32),
            dma, dma, dma, dma,
            [dma] * NBUF,
            [[dma] * NBUF, [dma] * NBUF],
        ],
    )
    return sca, scb


def _tc_round():
    BN = 1024

    def body(sL_ref, sR_ref, hL_ref, hR_ref, eaL_ref, eaR_ref, degc_ref,
             wd_ref, ws_ref, we_ref, bm_ref, wih_ref, whh_ref, bih_ref,
             bhh_ref, outL_ref, outR_ref):
        h = jnp.concatenate([hL_ref[...], hR_ref[...]], axis=1)
        S = jnp.concatenate([sL_ref[...], sR_ref[...]], axis=1)
        Ea = jnp.concatenate([eaL_ref[...], eaR_ref[...]], axis=1)
        deg = degc_ref[...][:, 0:1]
        dn = (deg > 0).astype(_f32)
        inv = 1.0 / jnp.maximum(deg, 1.0)
        ha = jnp.dot(h, wd_ref[...], preferred_element_type=_f32) + bm_ref[...]
        tot = (jnp.dot(S, ws_ref[...], preferred_element_type=_f32)
               + jnp.dot(Ea, we_ref[...], preferred_element_type=_f32))
        a = dn * ha + inv * tot
        gi = jnp.dot(a, wih_ref[...], preferred_element_type=_f32) + bih_ref[...]
        gh = jnp.dot(h, whh_ref[...], preferred_element_type=_f32) + bhh_ref[...]
        r = jax.nn.sigmoid(gi[:, :H] + gh[:, :H])
        z = jax.nn.sigmoid(gi[:, H:2 * H] + gh[:, H:2 * H])
        n = jnp.tanh(gi[:, 2 * H:] + r * gh[:, 2 * H:])
        out = (1.0 - z) * n + z * h
        outL_ref[...] = out[:, :HH]
        outR_ref[...] = out[:, HH:]

    half_spec = pl.BlockSpec((BN, HH), lambda i: (i, 0))
    deg_spec = pl.BlockSpec((BN, LANES), lambda i: (i, 0))

    def w_spec(r, c):
        return pl.BlockSpec((r, c), lambda i: (0, 0))

    in_specs = [half_spec] * 6 + [
        deg_spec,
        w_spec(H, 2 * H), w_spec(H, 2 * H), w_spec(H, 2 * H),
        w_spec(1, 2 * H),
        w_spec(2 * H, 3 * H), w_spec(H, 3 * H),
        w_spec(1, 3 * H), w_spec(1, 3 * H),
    ]

    half_t = jax.ShapeDtypeStruct((NPAD, HH), _f32)
    return pl.pallas_call(
        body,
        grid=(NPAD // BN,),
        in_specs=in_specs,
        out_specs=(half_spec, half_spec),
        out_shape=(half_t, half_t),
    )


_tc_call = _tc_round()


def kernel(x, edge_index, edge_attr, W_msg, b_msg, W_ih, W_hh, b_ih, b_hh):
    src2 = edge_index[0].reshape(CHROWS, CH)
    dst2 = edge_index[1].reshape(CHROWS, CH)
    dsta = edge_index[1].reshape(CHAROWS, CHA)
    h0 = jnp.zeros((NPAD, H), _f32).at[:N].set(x)
    hL, hR = h0[:, :HH], h0[:, HH:]

    sca_call, scb_call = _sc_calls()
    eaL, eaR, degc = sca_call(edge_attr, dsta)

    def round_weights(t):
        wmT = W_msg[t].T
        return (wmT[:H], wmT[H:2 * H], wmT[2 * H:],
                b_msg[t][None, :], W_ih[t].T, W_hh[t].T,
                b_ih[t][None, :], b_hh[t][None, :])

    for t in range(2):
        sL, sR = scb_call(hL, hR, src2, dst2)
        hL, hR = _tc_call(sL, sR, hL, hR, eaL, eaR, degc, *round_weights(t))

    return jnp.concatenate([hL, hR], axis=1)[:N]


# speedup vs baseline: 1.0692x; 1.0692x over previous
import functools

import jax
import jax.numpy as jnp
from jax import lax
from jax.experimental import pallas as pl
from jax.experimental.pallas import tpu as pltpu
from jax.experimental.pallas import tpu_sc as plsc

N = 10000
E = 320000
H = 128
HH = H // 2
NPAD = 10240
ROWS_PER_TILE = NPAD // 16
CH = 80
LANES = 16
EPT = E // 16
NCHUNKS = EPT // CH

_f32 = jnp.float32


def _fill_const(ref, nrows, ncols, val):
    v16 = jnp.full((LANES,), val, _f32)

    def body(r, carry):
        for cc in range(ncols // LANES):
            ref[r, pl.ds(cc * LANES, LANES)] = v16
        return carry

    lax.fori_loop(0, nrows, body, None)


def _zero_acc(acc, zbuf, tid, zrows):
    def body(j, carry):
        pltpu.sync_copy(zbuf,
                        acc.at[pl.ds(tid * ROWS_PER_TILE + j * zrows, zrows)])
        return carry

    lax.fori_loop(0, ROWS_PER_TILE // zrows, body, None)


NBUF = 5
NGRP = NCHUNKS // NBUF
CHROWS = E // CH
CHA = 80
CHAROWS = E // CHA
NCHUNKS_A = EPT // CHA
NGRP_A = NCHUNKS_A // NBUF
ZROWS = 64


def _sc_pass_a(ea_hbm, dsta_hbm, eaL_out, eaR_out, degc_out,
               idx_d, erows, ones_v, zbuf, zd, eacc, degacc,
               sem_i0, sem_i1, sems_r, sems_w, sem_dg0):
    cid = lax.axis_index("c")
    tid = lax.axis_index("s")

    _fill_const(zbuf, ZROWS, HH, 0.0)
    _fill_const(zd, 128, LANES, 0.0)
    _fill_const(ones_v, CHA, LANES, 1.0)
    _zero_acc(eacc, zbuf, tid, ZROWS)

    @pl.when(cid == 0)
    def _():
        def zero_deg(j, carry):
            pltpu.sync_copy(zd, degacc.at[pl.ds(tid * ROWS_PER_TILE + j * 128, 128)])
            return carry

        lax.fori_loop(0, ROWS_PER_TILE // 128, zero_deg, None)

    plsc.subcore_barrier()

    sem_i = (sem_i0, sem_i1)
    irow0 = tid * NCHUNKS_A

    def fire_idx(g, s):
        pltpu.async_copy(dsta_hbm.at[pl.ds(irow0 + g * NBUF, NBUF)],
                         idx_d.at[s], sem_i[s])

    fire_idx(0, 0)

    def group(g, s):
        @pl.when(g + 1 < NGRP_A)
        def _():
            fire_idx(g + 1, 1 - s)

        pltpu.make_async_copy(dsta_hbm.at[pl.ds(irow0, NBUF)],
                              idx_d.at[s], sem_i[s]).wait()
        for b in range(NBUF):
            gch = tid * EPT + (g * NBUF + b) * CHA

            @pl.when(cid == 0)
            def _(gch=gch, b=b):
                pltpu.async_copy(ea_hbm.at[pl.ds(gch, CHA), pl.ds(0, HH)],
                                 erows.at[b], sems_r[b])
                pltpu.async_copy(ones_v, degacc.at[idx_d.at[s, b]], sem_dg0,
                                 add=True)

            @pl.when(cid == 1)
            def _(gch=gch, b=b):
                pltpu.async_copy(ea_hbm.at[pl.ds(gch, CHA), pl.ds(HH, HH)],
                                 erows.at[b], sems_r[b])

        for b in range(NBUF):
            pltpu.make_async_copy(ea_hbm.at[pl.ds(0, CHA), pl.ds(0, HH)],
                                  erows.at[b], sems_r[b]).wait()
            pltpu.async_copy(erows.at[b], eacc.at[idx_d.at[s, b]],
                             sems_w[b], add=True)
        for b in range(NBUF):
            pltpu.make_async_copy(erows.at[b], eacc.at[pl.ds(0, CHA)],
                                  sems_w[b]).wait()

        @pl.when(cid == 0)
        def _():
            for b in range(NBUF):
                pltpu.make_async_copy(ones_v, degacc.at[pl.ds(0, CHA)],
                                      sem_dg0).wait()

    def outer(o, carry):
        group(2 * o, 0)
        group(2 * o + 1, 1)
        return carry

    lax.fori_loop(0, NGRP_A // 2, outer, None)
    plsc.subcore_barrier()
    sl = pl.ds(tid * ROWS_PER_TILE, ROWS_PER_TILE)

    @pl.when(cid == 0)
    def _():
        pltpu.sync_copy(eacc.at[sl], eaL_out.at[sl])
        pltpu.sync_copy(degacc.at[sl], degc_out.at[sl])

    @pl.when(cid == 1)
    def _():
        pltpu.sync_copy(eacc.at[sl], eaR_out.at[sl])


def _sc_pass_b(hL_hbm, hR_hbm, src2_hbm, dst2_hbm, sL_out, sR_out,
               idx_s, idx_d, rows, zbuf, acc,
               sem_is0, sem_is1, sem_id0, sem_id1, sems_g, sems_w):
    cid = lax.axis_index("c")
    tid = lax.axis_index("s")

    _fill_const(zbuf, ZROWS, HH, 0.0)
    _zero_acc(acc, zbuf, tid, ZROWS)
    plsc.subcore_barrier()

    sem_is = (sem_is0, sem_is1)
    sem_id = (sem_id0, sem_id1)
    irow0 = tid * NCHUNKS

    def fire_idx(g, s):
        sl_i = pl.ds(irow0 + g * NBUF, NBUF)
        pltpu.async_copy(src2_hbm.at[sl_i], idx_s.at[s], sem_is[s])
        pltpu.async_copy(dst2_hbm.at[sl_i], idx_d.at[s], sem_id[s])

    def drain_scatters(s):
        for b in range(NBUF):
            pltpu.make_async_copy(rows.at[s, b], acc.at[pl.ds(0, CH)],
                                  sems_w[s][b]).wait()

    def group(g, s):
        sl_i = pl.ds(irow0, NBUF)
        pltpu.make_async_copy(src2_hbm.at[sl_i], idx_s.at[s], sem_is[s]).wait()
        pltpu.make_async_copy(dst2_hbm.at[sl_i], idx_d.at[s], sem_id[s]).wait()
        for b in range(NBUF):
            @pl.when(cid == 0)
            def _(b=b):
                pltpu.async_copy(hL_hbm.at[idx_s.at[s, b]], rows.at[s, b],
                                 sems_g[b])

            @pl.when(cid == 1)
            def _(b=b):
                pltpu.async_copy(hR_hbm.at[idx_s.at[s, b]], rows.at[s, b],
                                 sems_g[b])

        for b in range(NBUF):
            pltpu.make_async_copy(hL_hbm.at[idx_s.at[s, b]], rows.at[s, b],
                                  sems_g[b]).wait()
            pltpu.async_copy(rows.at[s, b], acc.at[idx_d.at[s, b]],
                             sems_w[s][b], add=True)

        @pl.when(g >= 1)
        def _():
            drain_scatters(1 - s)

        @pl.when(g + 1 < NGRP)
        def _():
            fire_idx(g + 1, 1 - s)

    fire_idx(0, 0)

    def outer(o, carry):
        group(2 * o, 0)
        group(2 * o + 1, 1)
        return carry

    lax.fori_loop(0, NGRP // 2, outer, None)
    drain_scatters((NGRP - 1) % 2)
    plsc.subcore_barrier()
    sl = pl.ds(tid * ROWS_PER_TILE, ROWS_PER_TILE)

    @pl.when(cid == 0)
    def _():
        pltpu.sync_copy(acc.at[sl], sL_out.at[sl])

    @pl.when(cid == 1)
    def _():
        pltpu.sync_copy(acc.at[sl], sR_out.at[sl])


@functools.lru_cache(maxsize=1)
def _sc_calls():
    mesh = plsc.VectorSubcoreMesh(core_axis_name="c", subcore_axis_name="s")
    params = pltpu.CompilerParams(use_tc_tiling_on_sc=False)
    half = jax.ShapeDtypeStruct((NPAD, HH), _f32)
    dma = pltpu.SemaphoreType.DMA
    sca = pl.kernel(
        _sc_pass_a,
        out_type=(half, half, jax.ShapeDtypeStruct((NPAD, LANES), _f32)),
        mesh=mesh,
        compiler_params=params,
        scratch_types=[
            pltpu.VMEM((2, NBUF, CHA), jnp.int32),
            pltpu.VMEM((NBUF, CHA, HH), _f32),
            pltpu.VMEM((CHA, LANES), _f32),
            pltpu.VMEM((ZROWS, HH), _f32),
            pltpu.VMEM((128, LANES), _f32),
            pltpu.VMEM_SHARED((NPAD, HH), _f32),
            pltpu.VMEM_SHARED((NPAD, LANES), _f32),
            dma, dma,
            [dma] * NBUF,
            [dma] * NBUF,
            dma,
        ],
    )
    scb = pl.kernel(
        _sc_pass_b,
        out_type=(half, half),
        mesh=mesh,
        compiler_params=params,
        scratch_types=[
            pltpu.VMEM((2, NBUF, CH), jnp.int32),
            pltpu.VMEM((2, NBUF, CH), jnp.int32),
            pltpu.VMEM((2, NBUF, CH, HH), _f32),
            pltpu.VMEM((ZROWS, HH), _f32),
            pltpu.VMEM_SHARED((NPAD, HH), _f32),
            dma, dma, dma, dma,
            [dma] * NBUF,
            [[dma] * NBUF, [dma] * NBUF],
        ],
    )
    return sca, scb


def _tc_round(last):
    BN = 1024

    def body(sL_ref, sR_ref, hL_ref, hR_ref, eaL_ref, eaR_ref, degc_ref,
             wd_ref, ws_ref, we_ref, bm_ref, wih_ref, whh_ref, bih_ref,
             bhh_ref, *out_refs):
        h = jnp.concatenate([hL_ref[...], hR_ref[...]], axis=1)
        S = jnp.concatenate([sL_ref[...], sR_ref[...]], axis=1)
        Ea = jnp.concatenate([eaL_ref[...], eaR_ref[...]], axis=1)
        deg = degc_ref[...][:, 0:1]
        dn = (deg > 0).astype(_f32)
        inv = 1.0 / jnp.maximum(deg, 1.0)
        ha = jnp.dot(h, wd_ref[...], preferred_element_type=_f32) + bm_ref[...]
        tot = (jnp.dot(S, ws_ref[...], preferred_element_type=_f32)
               + jnp.dot(Ea, we_ref[...], preferred_element_type=_f32))
        a = dn * ha + inv * tot
        gi = jnp.dot(a, wih_ref[...], preferred_element_type=_f32) + bih_ref[...]
        gh = jnp.dot(h, whh_ref[...], preferred_element_type=_f32) + bhh_ref[...]
        r = jax.nn.sigmoid(gi[:, :H] + gh[:, :H])
        z = jax.nn.sigmoid(gi[:, H:2 * H] + gh[:, H:2 * H])
        n = jnp.tanh(gi[:, 2 * H:] + r * gh[:, 2 * H:])
        out = (1.0 - z) * n + z * h
        if last:
            out_refs[0][...] = out
        else:
            out_refs[0][...] = out[:, :HH]
            out_refs[1][...] = out[:, HH:]

    half_spec = pl.BlockSpec((BN, HH), lambda i: (i, 0))
    deg_spec = pl.BlockSpec((BN, LANES), lambda i: (i, 0))

    def w_spec(r, c):
        return pl.BlockSpec((r, c), lambda i: (0, 0))

    in_specs = [half_spec] * 6 + [
        deg_spec,
        w_spec(H, 2 * H), w_spec(H, 2 * H), w_spec(H, 2 * H),
        w_spec(1, 2 * H),
        w_spec(2 * H, 3 * H), w_spec(H, 3 * H),
        w_spec(1, 3 * H), w_spec(1, 3 * H),
    ]

    if last:
        out_specs = pl.BlockSpec((BN, H), lambda i: (i, 0))
        out_shape = jax.ShapeDtypeStruct((N, H), _f32)
    else:
        out_specs = (half_spec, half_spec)
        half_t = jax.ShapeDtypeStruct((N, HH), _f32)
        out_shape = (half_t, half_t)

    return pl.pallas_call(
        body,
        grid=(NPAD // BN,),
        in_specs=in_specs,
        out_specs=out_specs,
        out_shape=out_shape,
    )


_tc_mid = _tc_round(last=False)
_tc_last = _tc_round(last=True)


def kernel(x, edge_index, edge_attr, W_msg, b_msg, W_ih, W_hh, b_ih, b_hh):
    src2 = edge_index[0].reshape(CHROWS, CH)
    dst2 = edge_index[1].reshape(CHROWS, CH)
    hL, hR = x[:, :HH], x[:, HH:]

    sca_call, scb_call = _sc_calls()
    eaL, eaR, degc = sca_call(edge_attr, dst2)

    def round_weights(t):
        wmT = W_msg[t].T
        return (wmT[:H], wmT[H:2 * H], wmT[2 * H:],
                b_msg[t][None, :], W_ih[t].T, W_hh[t].T,
                b_ih[t][None, :], b_hh[t][None, :])

    s0L, s0R = scb_call(hL, hR, src2, dst2)
    hL, hR = _tc_mid(s0L, s0R, hL, hR, eaL, eaR, degc, *round_weights(0))
    s1L, s1R = scb_call(hL, hR, src2, dst2)
    return _tc_last(s1L, s1R, hL, hR, eaL, eaR, degc, *round_weights(1))


# speedup vs baseline: 1.0787x; 1.0089x over previous
import functools

import jax
import jax.numpy as jnp
from jax import lax
from jax.experimental import pallas as pl
from jax.experimental.pallas import tpu as pltpu
from jax.experimental.pallas import tpu_sc as plsc

N = 10000
E = 320000
H = 128
HH = H // 2
NPAD = 10240
ROWS_PER_TILE = NPAD // 16
CH = 80
LANES = 16
EPT = E // 16
NCHUNKS = EPT // CH

_f32 = jnp.float32


def _fill_const(ref, nrows, ncols, val):
    v16 = jnp.full((LANES,), val, _f32)

    def body(r, carry):
        for cc in range(ncols // LANES):
            ref[r, pl.ds(cc * LANES, LANES)] = v16
        return carry

    lax.fori_loop(0, nrows, body, None)


def _zero_acc(acc, zbuf, tid, zrows):
    def body(j, carry):
        pltpu.sync_copy(zbuf,
                        acc.at[pl.ds(tid * ROWS_PER_TILE + j * zrows, zrows)])
        return carry

    lax.fori_loop(0, ROWS_PER_TILE // zrows, body, None)


NBUF = 5
NGRP = NCHUNKS // NBUF
CHROWS = E // CH
CHA = 80
CHAROWS = E // CHA
NCHUNKS_A = EPT // CHA
NGRP_A = NCHUNKS_A // NBUF
ZROWS = 64


def _sc_pass_a(ea_hbm, dsta_hbm, eaL_out, eaR_out, degp_out,
               idx_d, erows, zbuf, degp, eacc,
               sem_i0, sem_i1, sems_r, sems_w):
    cid = lax.axis_index("c")
    tid = lax.axis_index("s")

    _fill_const(zbuf, ZROWS, HH, 0.0)
    _zero_acc(eacc, zbuf, tid, ZROWS)

    z16 = jnp.zeros((LANES,), _f32)

    @pl.when(cid == 0)
    def _():
        def zero_deg(k, carry):
            degp[pl.ds(k * LANES, LANES)] = z16
            return carry

        lax.fori_loop(0, NPAD // LANES, zero_deg, None)

    plsc.subcore_barrier()

    sem_i = (sem_i0, sem_i1)
    irow0 = tid * NCHUNKS_A
    ones16 = jnp.ones((LANES,), _f32)

    def fire_idx(g, s):
        pltpu.async_copy(dsta_hbm.at[pl.ds(irow0 + g * NBUF, NBUF)],
                         idx_d.at[s], sem_i[s])

    fire_idx(0, 0)

    def group(g, s):
        @pl.when(g + 1 < NGRP_A)
        def _():
            fire_idx(g + 1, 1 - s)

        pltpu.make_async_copy(dsta_hbm.at[pl.ds(irow0, NBUF)],
                              idx_d.at[s], sem_i[s]).wait()
        for b in range(NBUF):
            gch = tid * EPT + (g * NBUF + b) * CHA

            @pl.when(cid == 0)
            def _(gch=gch, b=b):
                pltpu.async_copy(ea_hbm.at[pl.ds(gch, CHA), pl.ds(0, HH)],
                                 erows.at[b], sems_r[b])

            @pl.when(cid == 1)
            def _(gch=gch, b=b):
                pltpu.async_copy(ea_hbm.at[pl.ds(gch, CHA), pl.ds(HH, HH)],
                                 erows.at[b], sems_r[b])

        @pl.when(cid == 0)
        def _():
            for b in range(NBUF):
                for j in range(CHA // LANES):
                    vec = idx_d[s, b, pl.ds(j * LANES, LANES)]
                    plsc.addupdate_scatter(degp, [vec], ones16)

        for b in range(NBUF):
            pltpu.make_async_copy(ea_hbm.at[pl.ds(0, CHA), pl.ds(0, HH)],
                                  erows.at[b], sems_r[b]).wait()
            pltpu.async_copy(erows.at[b], eacc.at[idx_d.at[s, b]],
                             sems_w[b], add=True)
        for b in range(NBUF):
            pltpu.make_async_copy(erows.at[b], eacc.at[pl.ds(0, CHA)],
                                  sems_w[b]).wait()

    def outer(o, carry):
        group(2 * o, 0)
        group(2 * o + 1, 1)
        return carry

    lax.fori_loop(0, NGRP_A // 2, outer, None)
    plsc.subcore_barrier()
    sl = pl.ds(tid * ROWS_PER_TILE, ROWS_PER_TILE)

    @pl.when(cid == 0)
    def _():
        pltpu.sync_copy(eacc.at[sl], eaL_out.at[sl])
        pltpu.sync_copy(degp, degp_out.at[tid])

    @pl.when(cid == 1)
    def _():
        pltpu.sync_copy(eacc.at[sl], eaR_out.at[sl])


def _sc_pass_b(hL_hbm, hR_hbm, src2_hbm, dst2_hbm, sL_out, sR_out,
               idx_s, idx_d, rows, zbuf, acc,
               sem_is0, sem_is1, sem_id0, sem_id1, sems_g, sems_w):
    cid = lax.axis_index("c")
    tid = lax.axis_index("s")

    _fill_const(zbuf, ZROWS, HH, 0.0)
    _zero_acc(acc, zbuf, tid, ZROWS)
    plsc.subcore_barrier()

    sem_is = (sem_is0, sem_is1)
    sem_id = (sem_id0, sem_id1)
    irow0 = tid * NCHUNKS

    def fire_idx(g, s):
        sl_i = pl.ds(irow0 + g * NBUF, NBUF)
        pltpu.async_copy(src2_hbm.at[sl_i], idx_s.at[s], sem_is[s])
        pltpu.async_copy(dst2_hbm.at[sl_i], idx_d.at[s], sem_id[s])

    def drain_scatters(s):
        for b in range(NBUF):
            pltpu.make_async_copy(rows.at[s, b], acc.at[pl.ds(0, CH)],
                                  sems_w[s][b]).wait()

    def group(g, s):
        sl_i = pl.ds(irow0, NBUF)
        pltpu.make_async_copy(src2_hbm.at[sl_i], idx_s.at[s], sem_is[s]).wait()
        pltpu.make_async_copy(dst2_hbm.at[sl_i], idx_d.at[s], sem_id[s]).wait()
        for b in range(NBUF):
            @pl.when(cid == 0)
            def _(b=b):
                pltpu.async_copy(hL_hbm.at[idx_s.at[s, b]], rows.at[s, b],
                                 sems_g[b])

            @pl.when(cid == 1)
            def _(b=b):
                pltpu.async_copy(hR_hbm.at[idx_s.at[s, b]], rows.at[s, b],
                                 sems_g[b])

        for b in range(NBUF):
            pltpu.make_async_copy(hL_hbm.at[idx_s.at[s, b]], rows.at[s, b],
                                  sems_g[b]).wait()
            pltpu.async_copy(rows.at[s, b], acc.at[idx_d.at[s, b]],
                             sems_w[s][b], add=True)

        @pl.when(g >= 1)
        def _():
            drain_scatters(1 - s)

        @pl.when(g + 1 < NGRP)
        def _():
            fire_idx(g + 1, 1 - s)

    fire_idx(0, 0)

    def outer(o, carry):
        group(2 * o, 0)
        group(2 * o + 1, 1)
        return carry

    lax.fori_loop(0, NGRP // 2, outer, None)
    drain_scatters((NGRP - 1) % 2)
    plsc.subcore_barrier()
    sl = pl.ds(tid * ROWS_PER_TILE, ROWS_PER_TILE)

    @pl.when(cid == 0)
    def _():
        pltpu.sync_copy(acc.at[sl], sL_out.at[sl])

    @pl.when(cid == 1)
    def _():
        pltpu.sync_copy(acc.at[sl], sR_out.at[sl])


@functools.lru_cache(maxsize=1)
def _sc_calls():
    mesh = plsc.VectorSubcoreMesh(core_axis_name="c", subcore_axis_name="s")
    params = pltpu.CompilerParams(use_tc_tiling_on_sc=False,
                                  needs_layout_passes=False)
    half = jax.ShapeDtypeStruct((NPAD, HH), _f32)
    dma = pltpu.SemaphoreType.DMA
    sca = pl.kernel(
        _sc_pass_a,
        out_type=(half, half, jax.ShapeDtypeStruct((16, NPAD), _f32)),
        mesh=mesh,
        compiler_params=params,
        scratch_types=[
            pltpu.VMEM((2, NBUF, CHA), jnp.int32),
            pltpu.VMEM((NBUF, CHA, HH), _f32),
            pltpu.VMEM((ZROWS, HH), _f32),
            pltpu.VMEM((NPAD,), _f32),
            pltpu.VMEM_SHARED((NPAD, HH), _f32),
            dma, dma,
            [dma] * NBUF,
            [dma] * NBUF,
        ],
    )
    scb = pl.kernel(
        _sc_pass_b,
        out_type=(half, half),
        mesh=mesh,
        compiler_params=params,
        scratch_types=[
            pltpu.VMEM((2, NBUF, CH), jnp.int32),
            pltpu.VMEM((2, NBUF, CH), jnp.int32),
            pltpu.VMEM((2, NBUF, CH, HH), _f32),
            pltpu.VMEM((ZROWS, HH), _f32),
            pltpu.VMEM_SHARED((NPAD, HH), _f32),
            dma, dma, dma, dma,
            [dma] * NBUF,
            [[dma] * NBUF, [dma] * NBUF],
        ],
    )
    return sca, scb


def _tc_round(last):
    BN = 1024

    def body(sL_ref, sR_ref, hL_ref, hR_ref, eaL_ref, eaR_ref, degc_ref,
             wd_ref, ws_ref, we_ref, bm_ref, wih_ref, whh_ref, bih_ref,
             bhh_ref, *out_refs):
        h = jnp.concatenate([hL_ref[...], hR_ref[...]], axis=1)
        S = jnp.concatenate([sL_ref[...], sR_ref[...]], axis=1)
        Ea = jnp.concatenate([eaL_ref[...], eaR_ref[...]], axis=1)
        deg = jnp.sum(degc_ref[...], axis=0)[:, None]
        dn = (deg > 0).astype(_f32)
        inv = 1.0 / jnp.maximum(deg, 1.0)
        ha = jnp.dot(h, wd_ref[...], preferred_element_type=_f32) + bm_ref[...]
        tot = (jnp.dot(S, ws_ref[...], preferred_element_type=_f32)
               + jnp.dot(Ea, we_ref[...], preferred_element_type=_f32))
        a = dn * ha + inv * tot
        gi = jnp.dot(a, wih_ref[...], preferred_element_type=_f32) + bih_ref[...]
        gh = jnp.dot(h, whh_ref[...], preferred_element_type=_f32) + bhh_ref[...]
        r = jax.nn.sigmoid(gi[:, :H] + gh[:, :H])
        z = jax.nn.sigmoid(gi[:, H:2 * H] + gh[:, H:2 * H])
        n = jnp.tanh(gi[:, 2 * H:] + r * gh[:, 2 * H:])
        out = (1.0 - z) * n + z * h
        if last:
            out_refs[0][...] = out
        else:
            out_refs[0][...] = out[:, :HH]
            out_refs[1][...] = out[:, HH:]

    half_spec = pl.BlockSpec((BN, HH), lambda i: (i, 0))
    deg_spec = pl.BlockSpec((16, BN), lambda i: (0, i))

    def w_spec(r, c):
        return pl.BlockSpec((r, c), lambda i: (0, 0))

    in_specs = [half_spec] * 6 + [
        deg_spec,
        w_spec(H, 2 * H), w_spec(H, 2 * H), w_spec(H, 2 * H),
        w_spec(1, 2 * H),
        w_spec(2 * H, 3 * H), w_spec(H, 3 * H),
        w_spec(1, 3 * H), w_spec(1, 3 * H),
    ]

    if last:
        out_specs = pl.BlockSpec((BN, H), lambda i: (i, 0))
        out_shape = jax.ShapeDtypeStruct((N, H), _f32)
    else:
        out_specs = (half_spec, half_spec)
        half_t = jax.ShapeDtypeStruct((N, HH), _f32)
        out_shape = (half_t, half_t)

    return pl.pallas_call(
        body,
        grid=(NPAD // BN,),
        in_specs=in_specs,
        out_specs=out_specs,
        out_shape=out_shape,
    )


_tc_mid = _tc_round(last=False)
_tc_last = _tc_round(last=True)


def kernel(x, edge_index, edge_attr, W_msg, b_msg, W_ih, W_hh, b_ih, b_hh):
    src2 = edge_index[0].reshape(CHROWS, CH)
    dst2 = edge_index[1].reshape(CHROWS, CH)
    hL, hR = x[:, :HH], x[:, HH:]

    sca_call, scb_call = _sc_calls()
    eaL, eaR, degc = sca_call(edge_attr, dst2)

    def round_weights(t):
        wmT = W_msg[t].T
        return (wmT[:H], wmT[H:2 * H], wmT[2 * H:],
                b_msg[t][None, :], W_ih[t].T, W_hh[t].T,
                b_ih[t][None, :], b_hh[t][None, :])

    s0L, s0R = scb_call(hL, hR, src2, dst2)
    hL, hR = _tc_mid(s0L, s0R, hL, hR, eaL, eaR, degc, *round_weights(0))
    s1L, s1R = scb_call(hL, hR, src2, dst2)
    return _tc_last(s1L, s1R, hL, hR, eaL, eaR, degc, *round_weights(1))
